# baseline (speedup 1.0000x reference)
import functools

import jax
import jax.numpy as jnp
from jax import lax
from jax.experimental import pallas as pl
from jax.experimental.pallas import tpu as pltpu
from jax.experimental.pallas import tpu_sc as plsc

_D = 128
_DF = 144
_CHUNK = 64
_NBUF = 4


@functools.lru_cache(maxsize=None)
def _make_sc_segsum(n_pad, e2_pad, nc, ns):
    nw = nc * ns
    per_tile = e2_pad // nw
    nchunks = per_tile // _CHUNK
    assert nchunks % _NBUF == 0 and nchunks >= 2 * _NBUF
    rows_per_tile = n_pad // ns
    zchunk = _CHUNK
    assert rows_per_tile % zchunk == 0

    mesh = plsc.VectorSubcoreMesh(core_axis_name="c", subcore_axis_name="s")

    @functools.partial(
        pl.kernel,
        mesh=mesh,
        out_type=[jax.ShapeDtypeStruct((n_pad, _DF), jnp.float32)
                  for _ in range(nc)],
        scratch_types=(
            [pltpu.VMEM((_CHUNK,), jnp.int32)] * _NBUF
            + [pltpu.VMEM((_CHUNK,), jnp.int32)] * _NBUF
            + [pltpu.VMEM((_CHUNK, _DF), jnp.float32)] * _NBUF
            + [pltpu.VMEM_SHARED((n_pad, _DF), jnp.float32)]
            + [pltpu.SemaphoreType.DMA] * (4 * _NBUF)
        ),
        compiler_params=pltpu.CompilerParams(use_tc_tiling_on_sc=False),
    )
    def segsum(faug_hbm, gid_hbm, seg_hbm, *out_and_scratch):
        outs = out_and_scratch[:nc]
        scratch = out_and_scratch[nc:]
        gibuf = scratch[:_NBUF]
        sibuf = scratch[_NBUF:2 * _NBUF]
        rows = scratch[2 * _NBUF:3 * _NBUF]
        acc_sh = scratch[3 * _NBUF]
        sems = scratch[3 * _NBUF + 1:]
        gisem = sems[:_NBUF]
        sisem = sems[_NBUF:2 * _NBUF]
        gsem = sems[2 * _NBUF:3 * _NBUF]
        ssem = sems[3 * _NBUF:]
        cid = lax.axis_index("c")
        sid = lax.axis_index("s")
        wid = cid * ns + sid

        def _idx_start(c, b):
            base = wid * per_tile + c * _CHUNK
            pltpu.async_copy(gid_hbm.at[pl.ds(base, _CHUNK)], gibuf[b], gisem[b])
            pltpu.async_copy(seg_hbm.at[pl.ds(base, _CHUNK)], sibuf[b], sisem[b])

        def _idx_wait(c, b):
            base = wid * per_tile + c * _CHUNK
            pltpu.make_async_copy(gid_hbm.at[pl.ds(base, _CHUNK)], gibuf[b], gisem[b]).wait()
            pltpu.make_async_copy(seg_hbm.at[pl.ds(base, _CHUNK)], sibuf[b], sisem[b]).wait()

        def _gather_start(b):
            pltpu.async_copy(faug_hbm.at[gibuf[b]], rows[b], gsem[b])

        def _gather_wait(b):
            pltpu.make_async_copy(faug_hbm.at[gibuf[b]], rows[b], gsem[b]).wait()

        def _scat_start(b):
            pltpu.async_copy(rows[b], acc_sh.at[sibuf[b]], ssem[b], add=True)

        def _scat_wait(b):
            pltpu.make_async_copy(rows[b], acc_sh.at[sibuf[b]], ssem[b]).wait()

        for b in range(_NBUF):
            _idx_start(b, b)

        def _zrow(r, carry):
            for j in range(_DF // 16):
                rows[0][r, pl.ds(j * 16, 16)] = jnp.zeros((16,), jnp.float32)
            return carry

        lax.fori_loop(0, zchunk, _zrow, 0)
        for k in range(rows_per_tile // zchunk):
            pltpu.async_copy(
                rows[0],
                acc_sh.at[pl.ds(sid * rows_per_tile + k * zchunk, zchunk)],
                ssem[k % _NBUF],
            )
        for k in range(rows_per_tile // zchunk):
            pltpu.make_async_copy(
                rows[0],
                acc_sh.at[pl.ds(sid * rows_per_tile + k * zchunk, zchunk)],
                ssem[k % _NBUF],
            ).wait()
        plsc.subcore_barrier()

        for b in range(_NBUF):
            _idx_wait(b, b)
            _gather_start(b)

        def _body(i, carry):
            c = _NBUF * i
            for b in range(_NBUF):
                _gather_wait(b)
                _idx_start(c + b + _NBUF, b)
                _scat_start(b)
            for b in range(_NBUF):
                _scat_wait(b)
                _idx_wait(c + b + _NBUF, b)
                _gather_start(b)
            return carry

        lax.fori_loop(0, nchunks // _NBUF - 1, _body, 0)

        for b in range(_NBUF):
            _gather_wait(b)
            _scat_start(b)
        for b in range(_NBUF):
            _scat_wait(b)

        plsc.subcore_barrier()
        for c in range(nc):
            @pl.when(cid == c)
            def _copy_out(c=c):
                pltpu.sync_copy(
                    acc_sh.at[pl.ds(sid * rows_per_tile, rows_per_tile)],
                    outs[c].at[pl.ds(sid * rows_per_tile, rows_per_tile)],
                )

    return segsum


@functools.lru_cache(maxsize=None)
def _make_mlp(n_out, nc, blk, d2, d3, dp):
    def body(x_ref, nb_ref, ops_ref, w1a_ref, w1b_ref, b1_ref, w2_ref,
             b2_ref, w3_ref, b3_ref, wp1_ref, bp1_ref, wp2_ref, bp2_ref, out_ref):
        nb = nb_ref[...]
        nb_sum = nb[:, :_D]
        deg = nb[:, _D:_D + 1]
        mean = nb_sum / jnp.maximum(deg, 1.0)
        x = x_ref[...]
        h = x @ w1a_ref[...] + mean @ w1b_ref[...] + b1_ref[...]
        h = jnp.maximum(h, 0.0)
        h = jnp.maximum(h @ w2_ref[...] + b2_ref[...], 0.0)
        g = h @ w3_ref[...] + b3_ref[...]
        pos = g[:, :3]
        nf = g[:, 3:]
        p = jnp.maximum(nf @ wp1_ref[...] + bp1_ref[...], 0.0)
        ip = jax.nn.sigmoid(p @ wp2_ref[...] + bp2_ref[...])
        ops = ops_ref[...]
        mx = jnp.max(ops, axis=1, keepdims=True)
        ex = jnp.exp(ops - mx)
        p0 = ex[:, 0:1] / jnp.sum(ex, axis=1, keepdims=True)
        m = jnp.logical_and(p0 > 0.5, deg > 0).astype(jnp.float32)
        out_ref[...] = jnp.concatenate([pos * m, nf * m, ip * m], axis=1)

    return pl.pallas_call(
        body,
        grid=(n_out // blk,),
        in_specs=[
            pl.BlockSpec((blk, _D), lambda i: (i, 0)),
            pl.BlockSpec((blk, _DF), lambda i: (i, 0)),
            pl.BlockSpec((blk, 4), lambda i: (i, 0)),
            pl.BlockSpec((_D, _D), lambda i: (0, 0)),
            pl.BlockSpec((_D, _D), lambda i: (0, 0)),
            pl.BlockSpec((1, _D), lambda i: (0, 0)),
            pl.BlockSpec((_D, d2), lambda i: (0, 0)),
            pl.BlockSpec((1, d2), lambda i: (0, 0)),
            pl.BlockSpec((d2, d3), lambda i: (0, 0)),
            pl.BlockSpec((1, d3), lambda i: (0, 0)),
            pl.BlockSpec((_D, dp), lambda i: (0, 0)),
            pl.BlockSpec((1, dp), lambda i: (0, 0)),
            pl.BlockSpec((dp, 1), lambda i: (0, 0)),
            pl.BlockSpec((1, 1), lambda i: (0, 0)),
        ],
        out_specs=pl.BlockSpec((blk, d3 + 1), lambda i: (i, 0)),
        out_shape=jax.ShapeDtypeStruct((n_out, d3 + 1), jnp.float32),
    )


def kernel(node_features, edge_index, node_operations, W1, b1, W2, b2, W3, b3,
           Wp1, bp1, Wp2, bp2):
    n, d = node_features.shape
    e = edge_index.shape[1]
    info = plsc.get_sparse_core_info()
    nc, ns = info.num_cores, info.num_subcores
    nw = nc * ns

    n_pad = ((n + 1023) // 1024) * 1024
    if n_pad == n:
        n_pad += 1024
    e2 = 2 * e
    per_tile = -(-e2 // (nw * _NBUF * _CHUNK)) * _NBUF * _CHUNK
    e2_pad = per_tile * nw
    padn = e2_pad - e2

    src = edge_index[0].astype(jnp.int32)
    dst = edge_index[1].astype(jnp.int32)
    ar = jnp.arange(padn, dtype=jnp.int32)
    seg = jnp.concatenate([src, dst, n + ar % (n_pad - n)])
    gid = jnp.concatenate([dst, src, ar % n])

    faug = jnp.concatenate(
        [
            jnp.concatenate(
                [node_features,
                 jnp.ones((n, 1), jnp.float32),
                 jnp.zeros((n, _DF - d - 1), jnp.float32)], axis=1),
            jnp.zeros((n_pad - n, _DF), jnp.float32),
        ],
        axis=0,
    )

    parts = _make_sc_segsum(n_pad, e2_pad, nc, ns)(faug, gid, seg)
    nb = parts[0]
    for c in range(1, nc):
        nb = nb + parts[c]

    d2 = W2.shape[1]
    d3 = W3.shape[1]
    dp = Wp1.shape[1]
    blk = 1000
    assert n % blk == 0
    mlp = _make_mlp(n, nc, blk, d2, d3, dp)
    return mlp(
        node_features, nb, node_operations,
        W1[:d], W1[d:], b1.reshape(1, -1),
        W2, b2.reshape(1, -1),
        W3, b3.reshape(1, -1),
        Wp1, bp1.reshape(1, -1),
        Wp2, bp2.reshape(1, -1),
    )


# speedup vs baseline: 1.0308x; 1.0308x over previous
import functools

import jax
import jax.numpy as jnp
from jax import lax
from jax.experimental import pallas as pl
from jax.experimental.pallas import tpu as pltpu
from jax.experimental.pallas import tpu_sc as plsc

_D = 128
_DF = 144
_CHUNK = 64
_NBUF = 4


@functools.lru_cache(maxsize=None)
def _make_sc_segsum(n_pad, e2_pad, nc, ns):
    nw = nc * ns
    per_tile = e2_pad // nw
    nchunks = per_tile // _CHUNK
    assert nchunks % _NBUF == 0 and nchunks >= 2 * _NBUF
    rows_per_tile = n_pad // ns
    zchunk = _CHUNK
    assert rows_per_tile % zchunk == 0

    mesh = plsc.VectorSubcoreMesh(core_axis_name="c", subcore_axis_name="s")

    @functools.partial(
        pl.kernel,
        mesh=mesh,
        out_type=jax.ShapeDtypeStruct((nc, n_pad, _DF), jnp.float32),
        scratch_types=(
            [pltpu.VMEM((_CHUNK,), jnp.int32)] * _NBUF
            + [pltpu.VMEM((_CHUNK,), jnp.int32)] * _NBUF
            + [pltpu.VMEM((_CHUNK, _DF), jnp.float32)] * _NBUF
            + [pltpu.VMEM_SHARED((n_pad, _DF), jnp.float32)]
            + [pltpu.SemaphoreType.DMA] * (4 * _NBUF)
        ),
        compiler_params=pltpu.CompilerParams(use_tc_tiling_on_sc=False),
    )
    def segsum(faug_hbm, gid_hbm, seg_hbm, out_hbm, *scratch):
        gibuf = scratch[:_NBUF]
        sibuf = scratch[_NBUF:2 * _NBUF]
        rows = scratch[2 * _NBUF:3 * _NBUF]
        acc_sh = scratch[3 * _NBUF]
        sems = scratch[3 * _NBUF + 1:]
        gisem = sems[:_NBUF]
        sisem = sems[_NBUF:2 * _NBUF]
        gsem = sems[2 * _NBUF:3 * _NBUF]
        ssem = sems[3 * _NBUF:]
        cid = lax.axis_index("c")
        sid = lax.axis_index("s")
        wid = cid * ns + sid

        def _idx_start(c, b):
            base = wid * per_tile + c * _CHUNK
            pltpu.async_copy(gid_hbm.at[pl.ds(base, _CHUNK)], gibuf[b], gisem[b])
            pltpu.async_copy(seg_hbm.at[pl.ds(base, _CHUNK)], sibuf[b], sisem[b])

        def _idx_wait(c, b):
            base = wid * per_tile + c * _CHUNK
            pltpu.make_async_copy(gid_hbm.at[pl.ds(base, _CHUNK)], gibuf[b], gisem[b]).wait()
            pltpu.make_async_copy(seg_hbm.at[pl.ds(base, _CHUNK)], sibuf[b], sisem[b]).wait()

        def _gather_start(b):
            pltpu.async_copy(faug_hbm.at[gibuf[b]], rows[b], gsem[b])

        def _gather_wait(b):
            pltpu.make_async_copy(faug_hbm.at[gibuf[b]], rows[b], gsem[b]).wait()

        def _scat_start(b):
            pltpu.async_copy(rows[b], acc_sh.at[sibuf[b]], ssem[b], add=True)

        def _scat_wait(b):
            pltpu.make_async_copy(rows[b], acc_sh.at[sibuf[b]], ssem[b]).wait()

        for b in range(_NBUF):
            _idx_start(b, b)

        def _zrow(r, carry):
            for j in range(_DF // 16):
                rows[0][r, pl.ds(j * 16, 16)] = jnp.zeros((16,), jnp.float32)
            return carry

        lax.fori_loop(0, zchunk, _zrow, 0)
        for k in range(rows_per_tile // zchunk):
            pltpu.async_copy(
                rows[0],
                acc_sh.at[pl.ds(sid * rows_per_tile + k * zchunk, zchunk)],
                ssem[k % _NBUF],
            )
        for k in range(rows_per_tile // zchunk):
            pltpu.make_async_copy(
                rows[0],
                acc_sh.at[pl.ds(sid * rows_per_tile + k * zchunk, zchunk)],
                ssem[k % _NBUF],
            ).wait()
        plsc.subcore_barrier()

        for b in range(_NBUF):
            _idx_wait(b, b)
            _gather_start(b)

        def _body(i, carry):
            c = _NBUF * i
            for b in range(_NBUF):
                _gather_wait(b)
                _idx_start(c + b + _NBUF, b)
                _scat_start(b)
            for b in range(_NBUF):
                _scat_wait(b)
                _idx_wait(c + b + _NBUF, b)
                _gather_start(b)
            return carry

        lax.fori_loop(0, nchunks // _NBUF - 1, _body, 0)

        for b in range(_NBUF):
            _gather_wait(b)
            _scat_start(b)
        for b in range(_NBUF):
            _scat_wait(b)

        plsc.subcore_barrier()
        pltpu.sync_copy(
            acc_sh.at[pl.ds(sid * rows_per_tile, rows_per_tile)],
            out_hbm.at[cid, pl.ds(sid * rows_per_tile, rows_per_tile)],
        )

    return segsum


@functools.lru_cache(maxsize=None)
def _make_mlp(n_out, nc, blk, d2, d3, dp):
    def body(x_ref, parts_ref, ops_ref, w1a_ref, w1b_ref, b1_ref, w2_ref,
             b2_ref, w3_ref, b3_ref, wp1_ref, bp1_ref, wp2_ref, bp2_ref, out_ref):
        nb = parts_ref[0]
        for c in range(1, nc):
            nb = nb + parts_ref[c]
        nb_sum = nb[:, :_D]
        deg = nb[:, _D:_D + 1]
        mean = nb_sum / jnp.maximum(deg, 1.0)
        x = x_ref[...]
        h = x @ w1a_ref[...] + mean @ w1b_ref[...] + b1_ref[...]
        h = jnp.maximum(h, 0.0)
        h = jnp.maximum(h @ w2_ref[...] + b2_ref[...], 0.0)
        g = h @ w3_ref[...] + b3_ref[...]
        pos = g[:, :3]
        nf = g[:, 3:]
        p = jnp.maximum(nf @ wp1_ref[...] + bp1_ref[...], 0.0)
        ip = jax.nn.sigmoid(p @ wp2_ref[...] + bp2_ref[...])
        ops = ops_ref[...]
        mx = jnp.max(ops, axis=1, keepdims=True)
        ex = jnp.exp(ops - mx)
        p0 = ex[:, 0:1] / jnp.sum(ex, axis=1, keepdims=True)
        m = jnp.logical_and(p0 > 0.5, deg > 0).astype(jnp.float32)
        out_ref[...] = jnp.concatenate([pos * m, nf * m, ip * m], axis=1)

    return pl.pallas_call(
        body,
        grid=(n_out // blk,),
        in_specs=[
            pl.BlockSpec((blk, _D), lambda i: (i, 0)),
            pl.BlockSpec((nc, blk, _DF), lambda i: (0, i, 0)),
            pl.BlockSpec((blk, 4), lambda i: (i, 0)),
            pl.BlockSpec((_D, _D), lambda i: (0, 0)),
            pl.BlockSpec((_D, _D), lambda i: (0, 0)),
            pl.BlockSpec((1, _D), lambda i: (0, 0)),
            pl.BlockSpec((_D, d2), lambda i: (0, 0)),
            pl.BlockSpec((1, d2), lambda i: (0, 0)),
            pl.BlockSpec((d2, d3), lambda i: (0, 0)),
            pl.BlockSpec((1, d3), lambda i: (0, 0)),
            pl.BlockSpec((_D, dp), lambda i: (0, 0)),
            pl.BlockSpec((1, dp), lambda i: (0, 0)),
            pl.BlockSpec((dp, 1), lambda i: (0, 0)),
            pl.BlockSpec((1, 1), lambda i: (0, 0)),
        ],
        out_specs=pl.BlockSpec((blk, d3 + 1), lambda i: (i, 0)),
        out_shape=jax.ShapeDtypeStruct((n_out, d3 + 1), jnp.float32),
    )


def kernel(node_features, edge_index, node_operations, W1, b1, W2, b2, W3, b3,
           Wp1, bp1, Wp2, bp2):
    n, d = node_features.shape
    e = edge_index.shape[1]
    info = plsc.get_sparse_core_info()
    nc, ns = info.num_cores, info.num_subcores
    nw = nc * ns

    n_pad = ((n + 1023) // 1024) * 1024
    if n_pad == n:
        n_pad += 1024
    e2 = 2 * e
    per_tile = -(-e2 // (nw * _NBUF * _CHUNK)) * _NBUF * _CHUNK
    e2_pad = per_tile * nw
    padn = e2_pad - e2

    src = edge_index[0].astype(jnp.int32)
    dst = edge_index[1].astype(jnp.int32)
    ar = jnp.arange(padn, dtype=jnp.int32)
    seg = jnp.concatenate([src, dst, n + ar % (n_pad - n)])
    gid = jnp.concatenate([dst, src, ar % n])

    faug = jnp.concatenate(
        [
            jnp.concatenate(
                [node_features,
                 jnp.ones((n, 1), jnp.float32),
                 jnp.zeros((n, _DF - d - 1), jnp.float32)], axis=1),
            jnp.zeros((n_pad - n, _DF), jnp.float32),
        ],
        axis=0,
    )

    parts = _make_sc_segsum(n_pad, e2_pad, nc, ns)(faug, gid, seg)

    d2 = W2.shape[1]
    d3 = W3.shape[1]
    dp = Wp1.shape[1]
    blk = 1000
    assert n % blk == 0
    mlp = _make_mlp(n, nc, blk, d2, d3, dp)
    return mlp(
        node_features, parts, node_operations,
        W1[:d], W1[d:], b1.reshape(1, -1),
        W2, b2.reshape(1, -1),
        W3, b3.reshape(1, -1),
        Wp1, bp1.reshape(1, -1),
        Wp2, bp2.reshape(1, -1),
    )
